# baseline (speedup 1.0000x reference)
import functools

import jax
import jax.numpy as jnp
from jax import lax
from jax.experimental import pallas as pl
from jax.experimental.pallas import tpu as pltpu
from jax.experimental.pallas import tpu_sc as plsc

N = 16384
D = 256
K = 8192
BR = 256
KC = 2048
COMMIT = 0.25


def _argmin_body(zsq_ref, wsq_ref, z_ref, w_ref, idx_ref):
    z = z_ref[...]
    zsq = zsq_ref[...]

    def chunk(c, carry):
        run_min, run_idx = carry
        w = w_ref[pl.ds(c * KC, KC), :]
        wsq = wsq_ref[:, pl.ds(c * KC, KC)]
        m = lax.dot_general(z, w, (((1,), (1,)), ((), ())),
                            preferred_element_type=jnp.float32)
        d = (zsq + wsq) - 2.0 * m
        cmin = jnp.min(d, axis=1, keepdims=True)
        iota = lax.broadcasted_iota(jnp.int32, (BR, KC), 1) + c * KC
        cidx = jnp.min(jnp.where(d == cmin, iota, jnp.int32(2**30)),
                       axis=1, keepdims=True)
        upd = cmin < run_min
        return (jnp.where(upd, cmin, run_min), jnp.where(upd, cidx, run_idx))

    init = (jnp.full((BR, 1), jnp.inf, jnp.float32),
            jnp.zeros((BR, 1), jnp.int32))
    _, run_idx = lax.fori_loop(0, K // KC, chunk, init)
    idx_ref[...] = run_idx


def _argmin_call(zsq, wsq2d, z_e, W):
    return pl.pallas_call(
        _argmin_body,
        grid=(N // BR,),
        in_specs=[
            pl.BlockSpec((BR, 1), lambda i: (i, 0)),
            pl.BlockSpec((1, K), lambda i: (0, 0)),
            pl.BlockSpec((BR, D), lambda i: (i, 0)),
            pl.BlockSpec((K, D), lambda i: (0, 0)),
        ],
        out_specs=pl.BlockSpec((BR, 1), lambda i: (i, 0)),
        out_shape=jax.ShapeDtypeStruct((N, 1), jnp.int32),
    )(zsq, wsq2d, z_e, W)


def _gather_call(W, idx_flat):
    info = plsc.get_sparse_core_info()
    nc, ns = info.num_cores, info.num_subcores
    nw = nc * ns
    bpw = N // nw
    ch = 128
    mesh = plsc.VectorSubcoreMesh(core_axis_name="c", subcore_axis_name="s")

    @functools.partial(
        pl.kernel, mesh=mesh,
        out_type=jax.ShapeDtypeStruct((N, D), jnp.float32),
        scratch_types=[
            pltpu.VMEM((ch,), jnp.int32),
            pltpu.VMEM((ch, D), jnp.float32),
            pltpu.SemaphoreType.DMA,
        ],
    )
    def gather(table_hbm, idx_hbm, out_hbm, idx_v, rows_v, sem):
        wid = lax.axis_index("s") * nc + lax.axis_index("c")

        def body(c, carry):
            base = wid * bpw + c * ch
            pltpu.sync_copy(idx_hbm.at[pl.ds(base, ch)], idx_v)
            pltpu.async_copy(table_hbm.at[idx_v], rows_v, sem).wait()
            pltpu.sync_copy(rows_v, out_hbm.at[pl.ds(base, ch)])
            return carry

        lax.fori_loop(0, bpw // ch, body, 0)

    return gather(W, idx_flat)


def _finalize_body(z_ref, q_ref, idx_ref, qst_ref, loss_ref, cnt_ref,
                   perp_ref):
    i = pl.program_id(0)

    @pl.when(i == 0)
    def _init():
        loss_ref[...] = jnp.zeros_like(loss_ref)
        cnt_ref[...] = jnp.zeros_like(cnt_ref)
        perp_ref[...] = jnp.zeros_like(perp_ref)

    z = z_ref[...]
    q = q_ref[...]
    diff = q - z
    qst_ref[...] = z + diff
    loss_ref[...] += jnp.sum(diff * diff, keepdims=True)
    idx = idx_ref[...]
    iota = lax.broadcasted_iota(jnp.int32, (BR, K), 1)
    onehot = (idx == iota).astype(jnp.float32)
    cnt_ref[...] += jnp.sum(onehot, axis=0, keepdims=True)

    @pl.when(i == (N // BR) - 1)
    def _fin():
        p = cnt_ref[...] * (1.0 / N)
        ent = jnp.sum(p * jnp.log(p + 1e-10), keepdims=True)
        perp_ref[...] = jnp.exp(-ent)


def _finalize_call(z_e, q, idx2d):
    return pl.pallas_call(
        _finalize_body,
        grid=(N // BR,),
        in_specs=[
            pl.BlockSpec((BR, D), lambda i: (i, 0)),
            pl.BlockSpec((BR, D), lambda i: (i, 0)),
            pl.BlockSpec((BR, 1), lambda i: (i, 0)),
        ],
        out_specs=[
            pl.BlockSpec((BR, D), lambda i: (i, 0)),
            pl.BlockSpec((1, 1), lambda i: (0, 0)),
            pl.BlockSpec((1, K), lambda i: (0, 0)),
            pl.BlockSpec((1, 1), lambda i: (0, 0)),
        ],
        out_shape=[
            jax.ShapeDtypeStruct((N, D), jnp.float32),
            jax.ShapeDtypeStruct((1, 1), jnp.float32),
            jax.ShapeDtypeStruct((1, K), jnp.float32),
            jax.ShapeDtypeStruct((1, 1), jnp.float32),
        ],
    )(z_e, q, idx2d)


def kernel(z_e, W):
    zsq = jnp.sum(z_e ** 2, axis=1, keepdims=True)
    wsq2d = jnp.sum(W ** 2, axis=1).reshape(1, K)
    idx2d = _argmin_call(zsq, wsq2d, z_e, W)
    q = _gather_call(W, idx2d.reshape(N))
    qst, loss_sum, _counts, perp = _finalize_call(z_e, q, idx2d)
    m = loss_sum[0, 0] / (N * D)
    loss = m + COMMIT * m
    return (loss, qst, perp[0, 0])


# speedup vs baseline: 1.4737x; 1.4737x over previous
import functools

import jax
import jax.numpy as jnp
from jax import lax
from jax.experimental import pallas as pl
from jax.experimental.pallas import tpu as pltpu
from jax.experimental.pallas import tpu_sc as plsc

N = 16384
D = 256
K = 8192
BR = 256
RH = 128
KC = 2048
COMMIT = 0.25


def _argmin_body(zsq_ref, wsq_ref, z_ref, w_ref, idx_ref):
    z2 = z_ref[...] * 2.0
    m2 = [
        lax.dot_general(z2, w_ref[pl.ds(c * KC, KC), :],
                        (((1,), (1,)), ((), ())),
                        preferred_element_type=jnp.float32)
        for c in range(K // KC)
    ]
    lanes = 128
    jtot = K // lanes
    jper = KC // lanes
    big = jnp.int32(2**30)
    for h in range(BR // RH):
        r0 = h * RH
        zsq = zsq_ref[pl.ds(r0, RH), :]
        run_v = jnp.full((RH, lanes), jnp.inf, jnp.float32)
        run_j = jnp.zeros((RH, lanes), jnp.int32)
        for j in range(jtot):
            a1 = zsq + wsq_ref[:, pl.ds(j * lanes, lanes)]
            jl = j % jper
            d = a1 - m2[j // jper][r0:r0 + RH, jl * lanes:(jl + 1) * lanes]
            upd = d < run_v
            run_v = jnp.where(upd, d, run_v)
            run_j = jnp.where(upd, jnp.int32(j), run_j)
        gmin = jnp.min(run_v, axis=1, keepdims=True)
        lane_iota = lax.broadcasted_iota(jnp.int32, (RH, lanes), 1)
        kc = jnp.where(run_v == gmin, run_j * lanes + lane_iota, big)
        idx_ref[pl.ds(r0, RH), :] = jnp.min(kc, axis=1, keepdims=True)


def _argmin_call(zsq, wsq2d, z_e, W):
    return pl.pallas_call(
        _argmin_body,
        grid=(N // BR,),
        in_specs=[
            pl.BlockSpec((BR, 1), lambda i: (i, 0)),
            pl.BlockSpec((1, K), lambda i: (0, 0)),
            pl.BlockSpec((BR, D), lambda i: (i, 0)),
            pl.BlockSpec((K, D), lambda i: (0, 0)),
        ],
        out_specs=pl.BlockSpec((BR, 1), lambda i: (i, 0)),
        out_shape=jax.ShapeDtypeStruct((N, 1), jnp.int32),
    )(zsq, wsq2d, z_e, W)


def _gather_call(W, idx_flat):
    info = plsc.get_sparse_core_info()
    nc, ns = info.num_cores, info.num_subcores
    nw = nc * ns
    bpw = N // nw
    ch = 128
    mesh = plsc.VectorSubcoreMesh(core_axis_name="c", subcore_axis_name="s")

    @functools.partial(
        pl.kernel, mesh=mesh,
        out_type=jax.ShapeDtypeStruct((N, D), jnp.float32),
        scratch_types=[
            pltpu.VMEM((ch,), jnp.int32),
            pltpu.VMEM((ch, D), jnp.float32),
            pltpu.SemaphoreType.DMA,
        ],
    )
    def gather(table_hbm, idx_hbm, out_hbm, idx_v, rows_v, sem):
        wid = lax.axis_index("s") * nc + lax.axis_index("c")

        def body(c, carry):
            base = wid * bpw + c * ch
            pltpu.sync_copy(idx_hbm.at[pl.ds(base, ch)], idx_v)
            pltpu.async_copy(table_hbm.at[idx_v], rows_v, sem).wait()
            pltpu.sync_copy(rows_v, out_hbm.at[pl.ds(base, ch)])
            return carry

        lax.fori_loop(0, bpw // ch, body, 0)

    return gather(W, idx_flat)


def _finalize_body(z_ref, q_ref, idx_ref, qst_ref, loss_ref, cnt_ref,
                   perp_ref):
    i = pl.program_id(0)

    @pl.when(i == 0)
    def _init():
        loss_ref[...] = jnp.zeros_like(loss_ref)
        cnt_ref[...] = jnp.zeros_like(cnt_ref)
        perp_ref[...] = jnp.zeros_like(perp_ref)

    z = z_ref[...]
    q = q_ref[...]
    diff = q - z
    qst_ref[...] = z + diff
    loss_ref[...] += jnp.sum(diff * diff, keepdims=True)
    idx = idx_ref[...]
    iota = lax.broadcasted_iota(jnp.int32, (BR, K), 1)
    onehot = (idx == iota).astype(jnp.float32)
    cnt_ref[...] += jnp.sum(onehot, axis=0, keepdims=True)

    @pl.when(i == (N // BR) - 1)
    def _fin():
        p = cnt_ref[...] * (1.0 / N)
        ent = jnp.sum(p * jnp.log(p + 1e-10), keepdims=True)
        perp_ref[...] = jnp.exp(-ent)


def _finalize_call(z_e, q, idx2d):
    return pl.pallas_call(
        _finalize_body,
        grid=(N // BR,),
        in_specs=[
            pl.BlockSpec((BR, D), lambda i: (i, 0)),
            pl.BlockSpec((BR, D), lambda i: (i, 0)),
            pl.BlockSpec((BR, 1), lambda i: (i, 0)),
        ],
        out_specs=[
            pl.BlockSpec((BR, D), lambda i: (i, 0)),
            pl.BlockSpec((1, 1), lambda i: (0, 0)),
            pl.BlockSpec((1, K), lambda i: (0, 0)),
            pl.BlockSpec((1, 1), lambda i: (0, 0)),
        ],
        out_shape=[
            jax.ShapeDtypeStruct((N, D), jnp.float32),
            jax.ShapeDtypeStruct((1, 1), jnp.float32),
            jax.ShapeDtypeStruct((1, K), jnp.float32),
            jax.ShapeDtypeStruct((1, 1), jnp.float32),
        ],
    )(z_e, q, idx2d)


def kernel(z_e, W):
    zsq = jnp.sum(z_e ** 2, axis=1, keepdims=True)
    wsq2d = jnp.sum(W ** 2, axis=1).reshape(1, K)
    idx2d = _argmin_call(zsq, wsq2d, z_e, W)
    q = _gather_call(W, idx2d.reshape(N))
    qst, loss_sum, _counts, perp = _finalize_call(z_e, q, idx2d)
    m = loss_sum[0, 0] / (N * D)
    loss = m + COMMIT * m
    return (loss, qst, perp[0, 0])


# speedup vs baseline: 1.8441x; 1.2513x over previous
import functools

import jax
import jax.numpy as jnp
from jax import lax
from jax.experimental import pallas as pl
from jax.experimental.pallas import tpu as pltpu
from jax.experimental.pallas import tpu_sc as plsc

N = 16384
D = 256
K = 8192
BR = 256
RH = 128
KC = 2048
COMMIT = 0.25


def _argmin_body(zsq_ref, wsq_ref, z_ref, w_ref, idx_ref):
    z2 = z_ref[...] * 2.0
    m2 = [
        lax.dot_general(z2, w_ref[pl.ds(c * KC, KC), :],
                        (((1,), (1,)), ((), ())),
                        preferred_element_type=jnp.float32)
        for c in range(K // KC)
    ]
    lanes = 128
    jtot = K // lanes
    jper = KC // lanes
    big = jnp.int32(2**30)
    for h in range(BR // RH):
        r0 = h * RH
        zsq = zsq_ref[pl.ds(r0, RH), :]
        run_v = jnp.full((RH, lanes), jnp.inf, jnp.float32)
        run_j = jnp.zeros((RH, lanes), jnp.int32)
        for j in range(jtot):
            a1 = zsq + wsq_ref[:, pl.ds(j * lanes, lanes)]
            jl = j % jper
            d = a1 - m2[j // jper][r0:r0 + RH, jl * lanes:(jl + 1) * lanes]
            upd = d < run_v
            run_v = jnp.where(upd, d, run_v)
            run_j = jnp.where(upd, jnp.int32(j), run_j)
        gmin = jnp.min(run_v, axis=1, keepdims=True)
        lane_iota = lax.broadcasted_iota(jnp.int32, (RH, lanes), 1)
        kc = jnp.where(run_v == gmin, run_j * lanes + lane_iota, big)
        idx_ref[pl.ds(r0, RH), :] = jnp.min(kc, axis=1, keepdims=True)


def _argmin_call(zsq, wsq2d, z_e, W):
    return pl.pallas_call(
        _argmin_body,
        grid=(N // BR,),
        in_specs=[
            pl.BlockSpec((BR, 1), lambda i: (i, 0)),
            pl.BlockSpec((1, K), lambda i: (0, 0)),
            pl.BlockSpec((BR, D), lambda i: (i, 0)),
            pl.BlockSpec((K, D), lambda i: (0, 0)),
        ],
        out_specs=pl.BlockSpec((BR, 1), lambda i: (i, 0)),
        out_shape=jax.ShapeDtypeStruct((N, 1), jnp.int32),
    )(zsq, wsq2d, z_e, W)


def _sc_fused_call(W, idx_flat, z_e, zeros_k):
    info = plsc.get_sparse_core_info()
    nc, ns = info.num_cores, info.num_subcores
    nw = nc * ns
    bpw = N // nw
    ch = 128
    mesh = plsc.VectorSubcoreMesh(core_axis_name="c", subcore_axis_name="s")

    @functools.partial(
        pl.kernel, mesh=mesh,
        out_type=[
            jax.ShapeDtypeStruct((N, D), jnp.float32),
            jax.ShapeDtypeStruct((nw, 16), jnp.float32),
            jax.ShapeDtypeStruct((nc, K), jnp.int32),
        ],
        scratch_types=[
            pltpu.VMEM((bpw,), jnp.int32),
            pltpu.VMEM((bpw,), jnp.int32),
            pltpu.VMEM((ch, D), jnp.float32),
            pltpu.VMEM((ch, D), jnp.float32),
            pltpu.VMEM((16,), jnp.float32),
            pltpu.VMEM_SHARED((K,), jnp.int32),
            pltpu.SemaphoreType.DMA,
        ],
    )
    def fused(table_hbm, idx_hbm, z_hbm, zero_hbm,
              qst_hbm, loss_hbm, cnt_hbm,
              idx_v, ones_v, rows_v, z_v, loss_v, cnt_sh, sem):
        cid = lax.axis_index("c")
        sid = lax.axis_index("s")
        wid = sid * nc + cid
        base = wid * bpw
        pltpu.sync_copy(idx_hbm.at[pl.ds(base, bpw)], idx_v)
        one16 = jnp.ones((16,), jnp.int32)
        for b in range(bpw // 16):
            ones_v[pl.ds(b * 16, 16)] = one16

        @pl.when(sid == 0)
        def _zero():
            pltpu.sync_copy(zero_hbm, cnt_sh)

        plsc.subcore_barrier()
        pltpu.sync_copy(ones_v, cnt_sh.at[idx_v], add=True)

        def chunk(c, acc):
            cbase = base + c * ch
            pltpu.async_copy(table_hbm.at[idx_v.at[pl.ds(c * ch, ch)]],
                             rows_v, sem).wait()
            pltpu.sync_copy(z_hbm.at[pl.ds(cbase, ch)], z_v)

            def row(r, acc):
                for l in range(D // 16):
                    q = rows_v[r, pl.ds(l * 16, 16)]
                    zz = z_v[r, pl.ds(l * 16, 16)]
                    dd = q - zz
                    rows_v[r, pl.ds(l * 16, 16)] = zz + dd
                    acc = acc + dd * dd
                return acc

            acc = lax.fori_loop(0, ch, row, acc)
            pltpu.sync_copy(rows_v, qst_hbm.at[pl.ds(cbase, ch)])
            return acc

        acc = lax.fori_loop(0, bpw // ch, chunk, jnp.zeros((16,), jnp.float32))
        loss_v[...] = acc
        pltpu.sync_copy(loss_v, loss_hbm.at[wid])
        plsc.subcore_barrier()

        @pl.when(sid == 0)
        def _out():
            pltpu.sync_copy(cnt_sh, cnt_hbm.at[cid])

    return fused(W, idx_flat, z_e, zeros_k)


def _finalize_body(lossp_ref, cnt_ref, loss_ref, perp_ref):
    m = jnp.sum(lossp_ref[...], keepdims=True) / (N * D)
    loss_ref[...] = m + COMMIT * m
    csum = cnt_ref[0:8, :] + cnt_ref[8:16, :]
    p = csum * (1.0 / N)
    ent = jnp.sum(p * jnp.log(p + 1e-10), keepdims=True)
    perp_ref[...] = jnp.exp(-ent)


def _finalize_call(lossp, cnt):
    return pl.pallas_call(
        _finalize_body,
        grid=(1,),
        in_specs=[
            pl.BlockSpec((4, 128), lambda i: (0, 0)),
            pl.BlockSpec((16, 1024), lambda i: (0, 0)),
        ],
        out_specs=[
            pl.BlockSpec((1, 1), lambda i: (0, 0)),
            pl.BlockSpec((1, 1), lambda i: (0, 0)),
        ],
        out_shape=[
            jax.ShapeDtypeStruct((1, 1), jnp.float32),
            jax.ShapeDtypeStruct((1, 1), jnp.float32),
        ],
    )(lossp, cnt)


def kernel(z_e, W):
    zsq = jnp.sum(z_e ** 2, axis=1, keepdims=True)
    wsq2d = jnp.sum(W ** 2, axis=1).reshape(1, K)
    idx2d = _argmin_call(zsq, wsq2d, z_e, W)
    zeros_k = jnp.zeros((K,), jnp.int32)
    qst, lossp, cnt = _sc_fused_call(W, idx2d.reshape(N), z_e, zeros_k)
    loss2d, perp2d = _finalize_call(
        lossp.reshape(4, 128), cnt.astype(jnp.float32).reshape(16, 1024))
    return (loss2d[0, 0], qst, perp2d[0, 0])


# speedup vs baseline: 1.9722x; 1.0694x over previous
import functools

import jax
import jax.numpy as jnp
from jax import lax
from jax.experimental import pallas as pl
from jax.experimental.pallas import tpu as pltpu
from jax.experimental.pallas import tpu_sc as plsc

N = 16384
D = 256
K = 8192
BR = 512
RH = 128
KC = 2048
COMMIT = 0.25


def _argmin_body(zsq_ref, wsq_ref, z_ref, w_ref, idx_ref):
    z2 = z_ref[...] * 2.0
    m2 = [
        lax.dot_general(z2, w_ref[pl.ds(c * KC, KC), :],
                        (((1,), (1,)), ((), ())),
                        preferred_element_type=jnp.float32)
        for c in range(K // KC)
    ]
    lanes = 128
    jtot = K // lanes
    jper = KC // lanes
    big = jnp.int32(2**30)
    for h in range(BR // RH):
        r0 = h * RH
        zsq = zsq_ref[pl.ds(r0, RH), :]
        run_v = jnp.full((RH, lanes), jnp.inf, jnp.float32)
        run_j = jnp.zeros((RH, lanes), jnp.int32)
        for j in range(jtot):
            a1 = zsq + wsq_ref[:, pl.ds(j * lanes, lanes)]
            jl = j % jper
            d = a1 - m2[j // jper][r0:r0 + RH, jl * lanes:(jl + 1) * lanes]
            upd = d < run_v
            run_v = jnp.where(upd, d, run_v)
            run_j = jnp.where(upd, jnp.int32(j), run_j)
        gmin = jnp.min(run_v, axis=1, keepdims=True)
        lane_iota = lax.broadcasted_iota(jnp.int32, (RH, lanes), 1)
        kc = jnp.where(run_v == gmin, run_j * lanes + lane_iota, big)
        idx_ref[pl.ds(r0, RH), :] = jnp.min(kc, axis=1, keepdims=True)


def _argmin_call(zsq, wsq2d, z_e, W):
    return pl.pallas_call(
        _argmin_body,
        grid=(N // BR,),
        in_specs=[
            pl.BlockSpec((BR, 1), lambda i: (i, 0)),
            pl.BlockSpec((1, K), lambda i: (0, 0)),
            pl.BlockSpec((BR, D), lambda i: (i, 0)),
            pl.BlockSpec((K, D), lambda i: (0, 0)),
        ],
        out_specs=pl.BlockSpec((BR, 1), lambda i: (i, 0)),
        out_shape=jax.ShapeDtypeStruct((N, 1), jnp.int32),
    )(zsq, wsq2d, z_e, W)


def _sc_fused_call(W, idx_flat, z_e, zeros_k):
    info = plsc.get_sparse_core_info()
    nc, ns = info.num_cores, info.num_subcores
    nw = nc * ns
    bpw = N // nw
    ch = 128
    mesh = plsc.VectorSubcoreMesh(core_axis_name="c", subcore_axis_name="s")

    @functools.partial(
        pl.kernel, mesh=mesh,
        out_type=[
            jax.ShapeDtypeStruct((N, D), jnp.float32),
            jax.ShapeDtypeStruct((nw, 16), jnp.float32),
            jax.ShapeDtypeStruct((nc, K), jnp.int32),
        ],
        scratch_types=[
            pltpu.VMEM((bpw,), jnp.int32),
            pltpu.VMEM((bpw,), jnp.int32),
            pltpu.VMEM((ch, D), jnp.float32),
            pltpu.VMEM((ch, D), jnp.float32),
            pltpu.VMEM((16,), jnp.float32),
            pltpu.VMEM_SHARED((K,), jnp.int32),
            pltpu.SemaphoreType.DMA,
        ],
    )
    def fused(table_hbm, idx_hbm, z_hbm, zero_hbm,
              qst_hbm, loss_hbm, cnt_hbm,
              idx_v, ones_v, rows_v, z_v, loss_v, cnt_sh, sem):
        cid = lax.axis_index("c")
        sid = lax.axis_index("s")
        wid = sid * nc + cid
        base = wid * bpw
        pltpu.sync_copy(idx_hbm.at[pl.ds(base, bpw)], idx_v)
        one16 = jnp.ones((16,), jnp.int32)
        for b in range(bpw // 16):
            ones_v[pl.ds(b * 16, 16)] = one16

        @pl.when(sid == 0)
        def _zero():
            pltpu.sync_copy(zero_hbm, cnt_sh)

        plsc.subcore_barrier()
        pltpu.sync_copy(ones_v, cnt_sh.at[idx_v], add=True)

        def chunk(c, acc):
            cbase = base + c * ch
            pltpu.async_copy(table_hbm.at[idx_v.at[pl.ds(c * ch, ch)]],
                             rows_v, sem).wait()
            pltpu.sync_copy(z_hbm.at[pl.ds(cbase, ch)], z_v)

            def row(r, acc):
                for l in range(D // 16):
                    q = rows_v[r, pl.ds(l * 16, 16)]
                    zz = z_v[r, pl.ds(l * 16, 16)]
                    dd = q - zz
                    rows_v[r, pl.ds(l * 16, 16)] = zz + dd
                    acc = acc + dd * dd
                return acc

            acc = lax.fori_loop(0, ch, row, acc)
            pltpu.sync_copy(rows_v, qst_hbm.at[pl.ds(cbase, ch)])
            return acc

        acc = lax.fori_loop(0, bpw // ch, chunk, jnp.zeros((16,), jnp.float32))
        loss_v[...] = acc
        pltpu.sync_copy(loss_v, loss_hbm.at[wid])
        plsc.subcore_barrier()

        @pl.when(sid == 0)
        def _out():
            pltpu.sync_copy(cnt_sh, cnt_hbm.at[cid])

    return fused(W, idx_flat, z_e, zeros_k)


def _finalize_body(lossp_ref, cnt_ref, loss_ref, perp_ref):
    m = jnp.sum(lossp_ref[...], keepdims=True) / (N * D)
    loss_ref[...] = m + COMMIT * m
    csum = cnt_ref[0:8, :] + cnt_ref[8:16, :]
    p = csum * (1.0 / N)
    ent = jnp.sum(p * jnp.log(p + 1e-10), keepdims=True)
    perp_ref[...] = jnp.exp(-ent)


def _finalize_call(lossp, cnt):
    return pl.pallas_call(
        _finalize_body,
        grid=(1,),
        in_specs=[
            pl.BlockSpec((4, 128), lambda i: (0, 0)),
            pl.BlockSpec((16, 1024), lambda i: (0, 0)),
        ],
        out_specs=[
            pl.BlockSpec((1, 1), lambda i: (0, 0)),
            pl.BlockSpec((1, 1), lambda i: (0, 0)),
        ],
        out_shape=[
            jax.ShapeDtypeStruct((1, 1), jnp.float32),
            jax.ShapeDtypeStruct((1, 1), jnp.float32),
        ],
    )(lossp, cnt)


def kernel(z_e, W):
    zsq = jnp.sum(z_e ** 2, axis=1, keepdims=True)
    wsq2d = jnp.sum(W ** 2, axis=1).reshape(1, K)
    idx2d = _argmin_call(zsq, wsq2d, z_e, W)
    zeros_k = jnp.zeros((K,), jnp.int32)
    qst, lossp, cnt = _sc_fused_call(W, idx2d.reshape(N), z_e, zeros_k)
    loss2d, perp2d = _finalize_call(
        lossp.reshape(4, 128), cnt.astype(jnp.float32).reshape(16, 1024))
    return (loss2d[0, 0], qst, perp2d[0, 0])


# speedup vs baseline: 2.0120x; 1.0202x over previous
import functools

import jax
import jax.numpy as jnp
from jax import lax
from jax.experimental import pallas as pl
from jax.experimental.pallas import tpu as pltpu
from jax.experimental.pallas import tpu_sc as plsc

N = 16384
D = 256
K = 8192
BR = 1024
RH = 128
KC = 2048
COMMIT = 0.25


def _argmin_body(zsq_ref, wsq_ref, z_ref, w_ref, idx_ref):
    z2 = z_ref[...] * 2.0
    m2 = [
        lax.dot_general(z2, w_ref[pl.ds(c * KC, KC), :],
                        (((1,), (1,)), ((), ())),
                        preferred_element_type=jnp.float32)
        for c in range(K // KC)
    ]
    lanes = 128
    jtot = K // lanes
    jper = KC // lanes
    big = jnp.int32(2**30)
    for h in range(BR // RH):
        r0 = h * RH
        zsq = zsq_ref[pl.ds(r0, RH), :]
        run_v = jnp.full((RH, lanes), jnp.inf, jnp.float32)
        run_j = jnp.zeros((RH, lanes), jnp.int32)
        for j in range(jtot):
            a1 = zsq + wsq_ref[:, pl.ds(j * lanes, lanes)]
            jl = j % jper
            d = a1 - m2[j // jper][r0:r0 + RH, jl * lanes:(jl + 1) * lanes]
            upd = d < run_v
            run_v = jnp.where(upd, d, run_v)
            run_j = jnp.where(upd, jnp.int32(j), run_j)
        gmin = jnp.min(run_v, axis=1, keepdims=True)
        lane_iota = lax.broadcasted_iota(jnp.int32, (RH, lanes), 1)
        kc = jnp.where(run_v == gmin, run_j * lanes + lane_iota, big)
        idx_ref[pl.ds(r0, RH), :] = jnp.min(kc, axis=1, keepdims=True)


def _argmin_call(zsq, wsq2d, z_e, W):
    return pl.pallas_call(
        _argmin_body,
        grid=(N // BR,),
        in_specs=[
            pl.BlockSpec((BR, 1), lambda i: (i, 0)),
            pl.BlockSpec((1, K), lambda i: (0, 0)),
            pl.BlockSpec((BR, D), lambda i: (i, 0)),
            pl.BlockSpec((K, D), lambda i: (0, 0)),
        ],
        out_specs=pl.BlockSpec((BR, 1), lambda i: (i, 0)),
        out_shape=jax.ShapeDtypeStruct((N, 1), jnp.int32),
    )(zsq, wsq2d, z_e, W)


def _sc_fused_call(W, idx_flat, z_e, zeros_k):
    info = plsc.get_sparse_core_info()
    nc, ns = info.num_cores, info.num_subcores
    nw = nc * ns
    bpw = N // nw
    ch = 128
    mesh = plsc.VectorSubcoreMesh(core_axis_name="c", subcore_axis_name="s")

    @functools.partial(
        pl.kernel, mesh=mesh,
        out_type=[
            jax.ShapeDtypeStruct((N, D), jnp.float32),
            jax.ShapeDtypeStruct((nw, 16), jnp.float32),
            jax.ShapeDtypeStruct((nc, K), jnp.int32),
        ],
        scratch_types=[
            pltpu.VMEM((bpw,), jnp.int32),
            pltpu.VMEM((bpw,), jnp.int32),
            pltpu.VMEM((ch, D), jnp.float32),
            pltpu.VMEM((ch, D), jnp.float32),
            pltpu.VMEM((16,), jnp.float32),
            pltpu.VMEM_SHARED((K,), jnp.int32),
            pltpu.SemaphoreType.DMA,
        ],
    )
    def fused(table_hbm, idx_hbm, z_hbm, zero_hbm,
              qst_hbm, loss_hbm, cnt_hbm,
              idx_v, ones_v, rows_v, z_v, loss_v, cnt_sh, sem):
        cid = lax.axis_index("c")
        sid = lax.axis_index("s")
        wid = sid * nc + cid
        base = wid * bpw
        pltpu.sync_copy(idx_hbm.at[pl.ds(base, bpw)], idx_v)
        one16 = jnp.ones((16,), jnp.int32)
        for b in range(bpw // 16):
            ones_v[pl.ds(b * 16, 16)] = one16

        @pl.when(sid == 0)
        def _zero():
            pltpu.sync_copy(zero_hbm, cnt_sh)

        plsc.subcore_barrier()
        pltpu.sync_copy(ones_v, cnt_sh.at[idx_v], add=True)

        def chunk(c, acc):
            cbase = base + c * ch
            pltpu.async_copy(table_hbm.at[idx_v.at[pl.ds(c * ch, ch)]],
                             rows_v, sem).wait()
            pltpu.sync_copy(z_hbm.at[pl.ds(cbase, ch)], z_v)

            def row(r, acc):
                for l in range(D // 16):
                    q = rows_v[r, pl.ds(l * 16, 16)]
                    zz = z_v[r, pl.ds(l * 16, 16)]
                    dd = q - zz
                    rows_v[r, pl.ds(l * 16, 16)] = zz + dd
                    acc = acc + dd * dd
                return acc

            acc = lax.fori_loop(0, ch, row, acc)
            pltpu.sync_copy(rows_v, qst_hbm.at[pl.ds(cbase, ch)])
            return acc

        acc = lax.fori_loop(0, bpw // ch, chunk, jnp.zeros((16,), jnp.float32))
        loss_v[...] = acc
        pltpu.sync_copy(loss_v, loss_hbm.at[wid])
        plsc.subcore_barrier()

        @pl.when(sid == 0)
        def _out():
            pltpu.sync_copy(cnt_sh, cnt_hbm.at[cid])

    return fused(W, idx_flat, z_e, zeros_k)


def _finalize_body(lossp_ref, cnt_ref, loss_ref, perp_ref):
    m = jnp.sum(lossp_ref[...], keepdims=True) / (N * D)
    loss_ref[...] = m + COMMIT * m
    csum = cnt_ref[0:8, :] + cnt_ref[8:16, :]
    p = csum * (1.0 / N)
    ent = jnp.sum(p * jnp.log(p + 1e-10), keepdims=True)
    perp_ref[...] = jnp.exp(-ent)


def _finalize_call(lossp, cnt):
    return pl.pallas_call(
        _finalize_body,
        grid=(1,),
        in_specs=[
            pl.BlockSpec((4, 128), lambda i: (0, 0)),
            pl.BlockSpec((16, 1024), lambda i: (0, 0)),
        ],
        out_specs=[
            pl.BlockSpec((1, 1), lambda i: (0, 0)),
            pl.BlockSpec((1, 1), lambda i: (0, 0)),
        ],
        out_shape=[
            jax.ShapeDtypeStruct((1, 1), jnp.float32),
            jax.ShapeDtypeStruct((1, 1), jnp.float32),
        ],
    )(lossp, cnt)


def kernel(z_e, W):
    zsq = jnp.sum(z_e ** 2, axis=1, keepdims=True)
    wsq2d = jnp.sum(W ** 2, axis=1).reshape(1, K)
    idx2d = _argmin_call(zsq, wsq2d, z_e, W)
    zeros_k = jnp.zeros((K,), jnp.int32)
    qst, lossp, cnt = _sc_fused_call(W, idx2d.reshape(N), z_e, zeros_k)
    loss2d, perp2d = _finalize_call(
        lossp.reshape(4, 128), cnt.astype(jnp.float32).reshape(16, 1024))
    return (loss2d[0, 0], qst, perp2d[0, 0])
